# baseline (speedup 1.0000x reference)
import functools

import jax
import jax.numpy as jnp
import numpy as np
from jax import lax
from jax.experimental import pallas as pl
from jax.experimental.pallas import tpu as pltpu
from jax.experimental.pallas import tpu_sc as plsc

N = 10000
E = 160000
D = 256
H = 256
OUT = 256
ALPHA = 0.1
THETA = 0.5
BETA1 = float(np.log(THETA / 1 + 1.0))
BETA2 = float(np.log(THETA / 2 + 1.0))

NC = 2
NS = 16
F = H // NC
K = 80
NCH = 128
EPT = K * NCH
E_PAD = NS * EPT
ACC_R = N + 8
RPT = 640
RPT_LAST = N - 15 * RPT

_sc_mesh = plsc.VectorSubcoreMesh(core_axis_name="c", subcore_axis_name="s")


@functools.partial(
    pl.kernel,
    out_type=[
        jax.ShapeDtypeStruct((N, F), jnp.float32),
        jax.ShapeDtypeStruct((N, F), jnp.float32),
    ],
    mesh=_sc_mesh,
    scratch_types=[
        pltpu.VMEM((EPT,), jnp.int32),
        pltpu.VMEM((EPT,), jnp.int32),
        pltpu.VMEM((K,), jnp.int32),
        pltpu.VMEM((K, F), jnp.float32),
        pltpu.VMEM((K, F), jnp.float32),
        pltpu.VMEM_SHARED((ACC_R, F), jnp.float32),
        pltpu.SemaphoreType.DMA,
        pltpu.SemaphoreType.DMA,
    ],
)
def _segsum_sc(src_hbm, dst_hbm, h_lo, h_hi, out_lo, out_hi,
               srcs, dsts, dstbuf, rows0, rows1, acc, sem0, sem1):
    c = lax.axis_index("c")
    s = lax.axis_index("s")
    rbase = pl.multiple_of(s * RPT, 8)

    def _zero_body(i, _):
        rows0[i // (F // 16), pl.ds((i % (F // 16)) * 16, 16)] = (
            jnp.zeros((16,), jnp.float32))
        return 0
    lax.fori_loop(0, K * (F // 16), _zero_body, 0)

    ebase = pl.multiple_of(s * EPT, 8)
    pltpu.async_copy(src_hbm.at[pl.ds(ebase, EPT)], srcs, sem0)
    pltpu.async_copy(dst_hbm.at[pl.ds(ebase, EPT)], dsts, sem1)

    @pl.when(s < 15)
    def _():
        for z in range(RPT // K):
            pltpu.sync_copy(rows0, acc.at[pl.ds(rbase + z * K, K)])

    @pl.when(s == 15)
    def _():
        for z in range(RPT_LAST // K):
            pltpu.sync_copy(rows0, acc.at[pl.ds(15 * RPT + z * K, K)])

    pltpu.make_async_copy(src_hbm.at[pl.ds(ebase, EPT)], srcs, sem0).wait()
    pltpu.make_async_copy(dst_hbm.at[pl.ds(ebase, EPT)], dsts, sem1).wait()
    plsc.subcore_barrier()

    def _start_gather(j, buf, sem):
        idx = srcs.at[pl.ds(pl.multiple_of(j * K, 8), K)]

        @pl.when(c == 0)
        def _():
            pltpu.async_copy(h_lo.at[idx], buf, sem)

        @pl.when(c == 1)
        def _():
            pltpu.async_copy(h_hi.at[idx], buf, sem)

    def _wait_gather(j, buf, sem):
        idx = srcs.at[pl.ds(pl.multiple_of(j * K, 8), K)]
        pltpu.make_async_copy(h_lo.at[idx], buf, sem).wait()

    def _stage_dst(j):
        for l in range(K // 16):
            dstbuf[pl.ds(l * 16, 16)] = dsts[pl.ds(j * K + l * 16, 16)]

    _start_gather(0, rows0, sem0)

    def _pipe(i, _):
        j0 = i * 2
        j1 = j0 + 1
        _start_gather(j1, rows1, sem1)
        _stage_dst(j0)
        _wait_gather(j0, rows0, sem0)
        pltpu.sync_copy(rows0, acc.at[dstbuf], add=True)

        @pl.when(j1 + 1 < NCH)
        def _():
            _start_gather(j1 + 1, rows0, sem0)
        _stage_dst(j1)
        _wait_gather(j1, rows1, sem1)
        pltpu.sync_copy(rows1, acc.at[dstbuf], add=True)
        return 0

    lax.fori_loop(0, NCH // 2, _pipe, 0)
    plsc.subcore_barrier()

    @pl.when(jnp.logical_and(c == 0, s < 15))
    def _():
        pltpu.sync_copy(acc.at[pl.ds(rbase, RPT)], out_lo.at[pl.ds(rbase, RPT)])

    @pl.when(jnp.logical_and(c == 0, s == 15))
    def _():
        pltpu.sync_copy(acc.at[pl.ds(15 * RPT, RPT_LAST)],
                        out_lo.at[pl.ds(15 * RPT, RPT_LAST)])

    @pl.when(jnp.logical_and(c == 1, s < 15))
    def _():
        pltpu.sync_copy(acc.at[pl.ds(rbase, RPT)], out_hi.at[pl.ds(rbase, RPT)])

    @pl.when(jnp.logical_and(c == 1, s == 15))
    def _():
        pltpu.sync_copy(acc.at[pl.ds(15 * RPT, RPT_LAST)],
                        out_hi.at[pl.ds(15 * RPT, RPT_LAST)])


BR = 1000


def _lin0_body(x_ref, w_ref, b_ref, h_ref, lo_ref, hi_ref):
    h = lax.dot_general(x_ref[...], w_ref[...], (((1,), (1,)), ((), ())),
                        preferred_element_type=jnp.float32)
    h = jnp.maximum(h + b_ref[...], 0.0)
    h_ref[...] = h
    lo_ref[...] = h[:, :F]
    hi_ref[...] = h[:, F:]


def _lin0_call(x, w, b):
    return pl.pallas_call(
        _lin0_body,
        grid=(N // BR,),
        in_specs=[
            pl.BlockSpec((BR, D), lambda i: (i, 0)),
            pl.BlockSpec((H, D), lambda i: (0, 0)),
            pl.BlockSpec((1, H), lambda i: (0, 0)),
        ],
        out_specs=[
            pl.BlockSpec((BR, H), lambda i: (i, 0)),
            pl.BlockSpec((BR, F), lambda i: (i, 0)),
            pl.BlockSpec((BR, F), lambda i: (i, 0)),
        ],
        out_shape=[
            jax.ShapeDtypeStruct((N, H), jnp.float32),
            jax.ShapeDtypeStruct((N, F), jnp.float32),
            jax.ShapeDtypeStruct((N, F), jnp.float32),
        ],
    )(x, w, b)


def _comb1_body(alo_ref, ahi_ref, x0_ref, w_ref, lo_ref, hi_ref):
    agg = jnp.concatenate([alo_ref[...], ahi_ref[...]], axis=1)
    u = (1.0 - ALPHA) * agg + ALPHA * x0_ref[...]
    o = (1.0 - BETA1) * u + BETA1 * jnp.dot(
        u, w_ref[...], preferred_element_type=jnp.float32)
    o = jnp.maximum(o, 0.0)
    lo_ref[...] = o[:, :F]
    hi_ref[...] = o[:, F:]


def _comb1_call(alo, ahi, x0, w):
    return pl.pallas_call(
        _comb1_body,
        grid=(N // BR,),
        in_specs=[
            pl.BlockSpec((BR, F), lambda i: (i, 0)),
            pl.BlockSpec((BR, F), lambda i: (i, 0)),
            pl.BlockSpec((BR, H), lambda i: (i, 0)),
            pl.BlockSpec((H, H), lambda i: (0, 0)),
        ],
        out_specs=[
            pl.BlockSpec((BR, F), lambda i: (i, 0)),
            pl.BlockSpec((BR, F), lambda i: (i, 0)),
        ],
        out_shape=[
            jax.ShapeDtypeStruct((N, F), jnp.float32),
            jax.ShapeDtypeStruct((N, F), jnp.float32),
        ],
    )(alo, ahi, x0, w)


def _comb2_body(alo_ref, ahi_ref, x0_ref, w_ref, w1_ref, b1_ref, out_ref):
    agg = jnp.concatenate([alo_ref[...], ahi_ref[...]], axis=1)
    u = (1.0 - ALPHA) * agg + ALPHA * x0_ref[...]
    o = (1.0 - BETA2) * u + BETA2 * jnp.dot(
        u, w_ref[...], preferred_element_type=jnp.float32)
    logits = lax.dot_general(o, w1_ref[...], (((1,), (1,)), ((), ())),
                             preferred_element_type=jnp.float32)
    out_ref[...] = logits + b1_ref[...]


def _comb2_call(alo, ahi, x0, w, w1, b1):
    return pl.pallas_call(
        _comb2_body,
        grid=(N // BR,),
        in_specs=[
            pl.BlockSpec((BR, F), lambda i: (i, 0)),
            pl.BlockSpec((BR, F), lambda i: (i, 0)),
            pl.BlockSpec((BR, H), lambda i: (i, 0)),
            pl.BlockSpec((H, H), lambda i: (0, 0)),
            pl.BlockSpec((OUT, H), lambda i: (0, 0)),
            pl.BlockSpec((1, OUT), lambda i: (0, 0)),
        ],
        out_specs=pl.BlockSpec((BR, OUT), lambda i: (i, 0)),
        out_shape=jax.ShapeDtypeStruct((N, OUT), jnp.float32),
    )(alo, ahi, x0, w, w1, b1)


def kernel(x, edge_index, lin0_W, lin0_b, conv_W1, conv_W2, lin1_W, lin1_b):
    pad = E_PAD - E
    src = jnp.concatenate([edge_index[0], jnp.zeros((pad,), jnp.int32)])
    dst = jnp.concatenate([edge_index[1], jnp.full((pad,), N, jnp.int32)])
    h, h_lo, h_hi = _lin0_call(x, lin0_W, lin0_b.reshape(1, H))
    a1_lo, a1_hi = _segsum_sc(src, dst, h_lo, h_hi)
    o1_lo, o1_hi = _comb1_call(a1_lo, a1_hi, h, conv_W1)
    a2_lo, a2_hi = _segsum_sc(src, dst, o1_lo, o1_hi)
    return _comb2_call(a2_lo, a2_hi, h, conv_W2, lin1_W, lin1_b.reshape(1, OUT))


# speedup vs baseline: 2.8101x; 2.8101x over previous
import functools

import jax
import jax.numpy as jnp
import numpy as np
from jax import lax
from jax.experimental import pallas as pl
from jax.experimental.pallas import tpu as pltpu
from jax.experimental.pallas import tpu_sc as plsc

N = 10000
E = 160000
D = 256
H = 256
OUT = 256
ALPHA = 0.1
THETA = 0.5
BETA1 = float(np.log(THETA / 1 + 1.0))
BETA2 = float(np.log(THETA / 2 + 1.0))

NC = 2
NS = 16
F = H // NC
EPT = E // NS
K = 80
NCH = EPT // K
NB = 3
RPT = 640
RPT_LAST = N - 15 * RPT

_sc_mesh = plsc.VectorSubcoreMesh(core_axis_name="c", subcore_axis_name="s")


@functools.partial(
    pl.kernel,
    out_type=[
        jax.ShapeDtypeStruct((N, F), jnp.float32),
        jax.ShapeDtypeStruct((N, F), jnp.float32),
    ],
    mesh=_sc_mesh,
    scratch_types=[
        pltpu.VMEM((EPT,), jnp.int32),
        pltpu.VMEM((EPT,), jnp.int32),
        pltpu.VMEM((K,), jnp.int32),
        pltpu.VMEM((K, F), jnp.float32),
        pltpu.VMEM((K, F), jnp.float32),
        pltpu.VMEM((K, F), jnp.float32),
        pltpu.VMEM_SHARED((N, F), jnp.float32),
        pltpu.SemaphoreType.DMA,
        pltpu.SemaphoreType.DMA,
        pltpu.SemaphoreType.DMA,
    ],
)
def _segsum_sc(src_hbm, dst_hbm, h_lo, h_hi, out_lo, out_hi,
               srcs, dsts, dstbuf, rows0, rows1, rows2, acc,
               sem0, sem1, sem2):
    c = lax.axis_index("c")
    s = lax.axis_index("s")
    rbase = pl.multiple_of(s * RPT, 8)

    ebase = pl.multiple_of(s * EPT, 8)
    pltpu.async_copy(src_hbm.at[pl.ds(ebase, EPT)], srcs, sem0)
    pltpu.async_copy(dst_hbm.at[pl.ds(ebase, EPT)], dsts, sem1)

    def _zero_body(i, _):
        rows0[i // (F // 16), pl.ds((i % (F // 16)) * 16, 16)] = (
            jnp.zeros((16,), jnp.float32))
        return 0
    lax.fori_loop(0, K * (F // 16), _zero_body, 0)

    @pl.when(s < 15)
    def _():
        for z in range(RPT // K):
            pltpu.sync_copy(rows0, acc.at[pl.ds(rbase + z * K, K)])

    @pl.when(s == 15)
    def _():
        for z in range(RPT_LAST // K):
            pltpu.sync_copy(rows0, acc.at[pl.ds(15 * RPT + z * K, K)])

    pltpu.make_async_copy(src_hbm.at[pl.ds(ebase, EPT)], srcs, sem0).wait()
    pltpu.make_async_copy(dst_hbm.at[pl.ds(ebase, EPT)], dsts, sem1).wait()
    plsc.subcore_barrier()

    def _start_gather(j, buf, sem):
        idx = srcs.at[pl.ds(pl.multiple_of(j * K, 8), K)]

        @pl.when(c == 0)
        def _():
            pltpu.async_copy(h_lo.at[idx], buf, sem)

        @pl.when(c == 1)
        def _():
            pltpu.async_copy(h_hi.at[idx], buf, sem)

    def _wait_gather(j, buf, sem):
        idx = srcs.at[pl.ds(pl.multiple_of(j * K, 8), K)]
        pltpu.make_async_copy(h_lo.at[idx], buf, sem).wait()

    def _stage_dst(j):
        for l in range(K // 16):
            dstbuf[pl.ds(l * 16, 16)] = dsts[pl.ds(j * K + l * 16, 16)]

    bufs = [(rows0, sem0), (rows1, sem1), (rows2, sem2)]
    for t in range(NB):
        _start_gather(t, bufs[t][0], bufs[t][1])

    def _pipe(i, _):
        for t in range(NB):
            j = i * NB + t
            _stage_dst(j)
            _wait_gather(j, bufs[t][0], bufs[t][1])

            @pl.when(j + NB < NCH)
            def _():
                _start_gather(j + NB, bufs[t][0], bufs[t][1])
            pltpu.sync_copy(bufs[t][0], acc.at[dstbuf], add=True)
        return 0

    lax.fori_loop(0, NCH // NB, _pipe, 0)
    for j in range(NB * (NCH // NB), NCH):
        t = j % NB
        _stage_dst(j)
        _wait_gather(j, bufs[t][0], bufs[t][1])
        pltpu.sync_copy(bufs[t][0], acc.at[dstbuf], add=True)
    plsc.subcore_barrier()

    @pl.when(jnp.logical_and(c == 0, s < 15))
    def _():
        pltpu.sync_copy(acc.at[pl.ds(rbase, RPT)], out_lo.at[pl.ds(rbase, RPT)])

    @pl.when(jnp.logical_and(c == 0, s == 15))
    def _():
        pltpu.sync_copy(acc.at[pl.ds(15 * RPT, RPT_LAST)],
                        out_lo.at[pl.ds(15 * RPT, RPT_LAST)])

    @pl.when(jnp.logical_and(c == 1, s < 15))
    def _():
        pltpu.sync_copy(acc.at[pl.ds(rbase, RPT)], out_hi.at[pl.ds(rbase, RPT)])

    @pl.when(jnp.logical_and(c == 1, s == 15))
    def _():
        pltpu.sync_copy(acc.at[pl.ds(15 * RPT, RPT_LAST)],
                        out_hi.at[pl.ds(15 * RPT, RPT_LAST)])


BR = 1000


def _lin0_body(x_ref, w_ref, b_ref, h_ref, lo_ref, hi_ref):
    h = lax.dot_general(x_ref[...], w_ref[...], (((1,), (1,)), ((), ())),
                        preferred_element_type=jnp.float32)
    h = jnp.maximum(h + b_ref[...], 0.0)
    h_ref[...] = h
    lo_ref[...] = h[:, :F]
    hi_ref[...] = h[:, F:]


def _lin0_call(x, w, b):
    return pl.pallas_call(
        _lin0_body,
        grid=(N // BR,),
        in_specs=[
            pl.BlockSpec((BR, D), lambda i: (i, 0)),
            pl.BlockSpec((H, D), lambda i: (0, 0)),
            pl.BlockSpec((1, H), lambda i: (0, 0)),
        ],
        out_specs=[
            pl.BlockSpec((BR, H), lambda i: (i, 0)),
            pl.BlockSpec((BR, F), lambda i: (i, 0)),
            pl.BlockSpec((BR, F), lambda i: (i, 0)),
        ],
        out_shape=[
            jax.ShapeDtypeStruct((N, H), jnp.float32),
            jax.ShapeDtypeStruct((N, F), jnp.float32),
            jax.ShapeDtypeStruct((N, F), jnp.float32),
        ],
    )(x, w, b)


def _comb1_body(alo_ref, ahi_ref, x0_ref, w_ref, lo_ref, hi_ref):
    agg = jnp.concatenate([alo_ref[...], ahi_ref[...]], axis=1)
    u = (1.0 - ALPHA) * agg + ALPHA * x0_ref[...]
    o = (1.0 - BETA1) * u + BETA1 * jnp.dot(
        u, w_ref[...], preferred_element_type=jnp.float32)
    o = jnp.maximum(o, 0.0)
    lo_ref[...] = o[:, :F]
    hi_ref[...] = o[:, F:]


def _comb1_call(alo, ahi, x0, w):
    return pl.pallas_call(
        _comb1_body,
        grid=(N // BR,),
        in_specs=[
            pl.BlockSpec((BR, F), lambda i: (i, 0)),
            pl.BlockSpec((BR, F), lambda i: (i, 0)),
            pl.BlockSpec((BR, H), lambda i: (i, 0)),
            pl.BlockSpec((H, H), lambda i: (0, 0)),
        ],
        out_specs=[
            pl.BlockSpec((BR, F), lambda i: (i, 0)),
            pl.BlockSpec((BR, F), lambda i: (i, 0)),
        ],
        out_shape=[
            jax.ShapeDtypeStruct((N, F), jnp.float32),
            jax.ShapeDtypeStruct((N, F), jnp.float32),
        ],
    )(alo, ahi, x0, w)


def _comb2_body(alo_ref, ahi_ref, x0_ref, w_ref, w1_ref, b1_ref, out_ref):
    agg = jnp.concatenate([alo_ref[...], ahi_ref[...]], axis=1)
    u = (1.0 - ALPHA) * agg + ALPHA * x0_ref[...]
    o = (1.0 - BETA2) * u + BETA2 * jnp.dot(
        u, w_ref[...], preferred_element_type=jnp.float32)
    logits = lax.dot_general(o, w1_ref[...], (((1,), (1,)), ((), ())),
                             preferred_element_type=jnp.float32)
    out_ref[...] = logits + b1_ref[...]


def _comb2_call(alo, ahi, x0, w, w1, b1):
    return pl.pallas_call(
        _comb2_body,
        grid=(N // BR,),
        in_specs=[
            pl.BlockSpec((BR, F), lambda i: (i, 0)),
            pl.BlockSpec((BR, F), lambda i: (i, 0)),
            pl.BlockSpec((BR, H), lambda i: (i, 0)),
            pl.BlockSpec((H, H), lambda i: (0, 0)),
            pl.BlockSpec((OUT, H), lambda i: (0, 0)),
            pl.BlockSpec((1, OUT), lambda i: (0, 0)),
        ],
        out_specs=pl.BlockSpec((BR, OUT), lambda i: (i, 0)),
        out_shape=jax.ShapeDtypeStruct((N, OUT), jnp.float32),
    )(alo, ahi, x0, w, w1, b1)


def kernel(x, edge_index, lin0_W, lin0_b, conv_W1, conv_W2, lin1_W, lin1_b):
    src = edge_index[0]
    dst = edge_index[1]
    h, h_lo, h_hi = _lin0_call(x, lin0_W, lin0_b.reshape(1, H))
    a1_lo, a1_hi = _segsum_sc(src, dst, h_lo, h_hi)
    o1_lo, o1_hi = _comb1_call(a1_lo, a1_hi, h, conv_W1)
    a2_lo, a2_hi = _segsum_sc(src, dst, o1_lo, o1_hi)
    return _comb2_call(a2_lo, a2_hi, h, conv_W2, lin1_W, lin1_b.reshape(1, OUT))


# speedup vs baseline: 2.8967x; 1.0308x over previous
import functools

import jax
import jax.numpy as jnp
import numpy as np
from jax import lax
from jax.experimental import pallas as pl
from jax.experimental.pallas import tpu as pltpu
from jax.experimental.pallas import tpu_sc as plsc

N = 10000
E = 160000
D = 256
H = 256
OUT = 256
ALPHA = 0.1
THETA = 0.5
BETA1 = float(np.log(THETA / 1 + 1.0))
BETA2 = float(np.log(THETA / 2 + 1.0))

NC = 2
NS = 16
F = H // NC
EPT = E // NS
K = 80
NCH = EPT // K
NB = 3
RPT = 640
RPT_LAST = N - 15 * RPT

_sc_mesh = plsc.VectorSubcoreMesh(core_axis_name="c", subcore_axis_name="s")


@functools.partial(
    pl.kernel,
    out_type=[
        jax.ShapeDtypeStruct((N, F), jnp.float32),
        jax.ShapeDtypeStruct((N, F), jnp.float32),
    ],
    mesh=_sc_mesh,
    scratch_types=[
        pltpu.VMEM((EPT,), jnp.int32),
        pltpu.VMEM((EPT,), jnp.int32),
        pltpu.VMEM((K,), jnp.int32),
        pltpu.VMEM((K, F), jnp.float32),
        pltpu.VMEM((K, F), jnp.float32),
        pltpu.VMEM((K, F), jnp.float32),
        pltpu.VMEM_SHARED((N, F), jnp.float32),
        pltpu.SemaphoreType.DMA,
        pltpu.SemaphoreType.DMA,
        pltpu.SemaphoreType.DMA,
    ],
)
def _segsum_sc(src_hbm, dst_hbm, h_lo, h_hi, out_lo, out_hi,
               srcs, dsts, dstbuf, rows0, rows1, rows2, acc,
               sem0, sem1, sem2):
    c = lax.axis_index("c")
    s = lax.axis_index("s")
    rbase = pl.multiple_of(s * RPT, 8)

    ebase = pl.multiple_of(s * EPT, 8)
    pltpu.async_copy(src_hbm.at[pl.ds(ebase, EPT)], srcs, sem0)
    pltpu.async_copy(dst_hbm.at[pl.ds(ebase, EPT)], dsts, sem1)

    def _zero_body(i, _):
        rows0[i // (F // 16), pl.ds((i % (F // 16)) * 16, 16)] = (
            jnp.zeros((16,), jnp.float32))
        return 0
    lax.fori_loop(0, K * (F // 16), _zero_body, 0)

    @pl.when(s < 15)
    def _():
        for z in range(RPT // K):
            pltpu.sync_copy(rows0, acc.at[pl.ds(rbase + z * K, K)])

    @pl.when(s == 15)
    def _():
        for z in range(RPT_LAST // K):
            pltpu.sync_copy(rows0, acc.at[pl.ds(15 * RPT + z * K, K)])

    pltpu.make_async_copy(src_hbm.at[pl.ds(ebase, EPT)], srcs, sem0).wait()
    pltpu.make_async_copy(dst_hbm.at[pl.ds(ebase, EPT)], dsts, sem1).wait()
    plsc.subcore_barrier()

    def _start_gather(j, buf, sem):
        idx = srcs.at[pl.ds(pl.multiple_of(j * K, 8), K)]

        @pl.when(c == 0)
        def _():
            pltpu.async_copy(h_lo.at[idx], buf, sem)

        @pl.when(c == 1)
        def _():
            pltpu.async_copy(h_hi.at[idx], buf, sem)

    def _wait_gather(j, buf, sem):
        idx = srcs.at[pl.ds(pl.multiple_of(j * K, 8), K)]
        pltpu.make_async_copy(h_lo.at[idx], buf, sem).wait()

    def _stage_dst(j):
        for l in range(K // 16):
            dstbuf[pl.ds(l * 16, 16)] = dsts[pl.ds(j * K + l * 16, 16)]

    bufs = [(rows0, sem0), (rows1, sem1), (rows2, sem2)]
    for t in range(NB):
        _start_gather(t, bufs[t][0], bufs[t][1])

    def _pipe(i, _):
        for t in range(NB):
            j = i * NB + t
            _stage_dst(j)
            _wait_gather(j, bufs[t][0], bufs[t][1])

            @pl.when(j + NB < NCH)
            def _():
                _start_gather(j + NB, bufs[t][0], bufs[t][1])
            pltpu.sync_copy(bufs[t][0], acc.at[dstbuf], add=True)
        return 0

    lax.fori_loop(0, NCH // NB, _pipe, 0)
    for j in range(NB * (NCH // NB), NCH):
        t = j % NB
        _stage_dst(j)
        _wait_gather(j, bufs[t][0], bufs[t][1])
        pltpu.sync_copy(bufs[t][0], acc.at[dstbuf], add=True)
    plsc.subcore_barrier()

    @pl.when(jnp.logical_and(c == 0, s < 15))
    def _():
        pltpu.sync_copy(acc.at[pl.ds(rbase, RPT)], out_lo.at[pl.ds(rbase, RPT)])

    @pl.when(jnp.logical_and(c == 0, s == 15))
    def _():
        pltpu.sync_copy(acc.at[pl.ds(15 * RPT, RPT_LAST)],
                        out_lo.at[pl.ds(15 * RPT, RPT_LAST)])

    @pl.when(jnp.logical_and(c == 1, s < 15))
    def _():
        pltpu.sync_copy(acc.at[pl.ds(rbase, RPT)], out_hi.at[pl.ds(rbase, RPT)])

    @pl.when(jnp.logical_and(c == 1, s == 15))
    def _():
        pltpu.sync_copy(acc.at[pl.ds(15 * RPT, RPT_LAST)],
                        out_hi.at[pl.ds(15 * RPT, RPT_LAST)])


BR = 2000


def _lin0_body(x_ref, w_ref, b_ref, lo_ref, hi_ref):
    h = lax.dot_general(x_ref[...], w_ref[...], (((1,), (1,)), ((), ())),
                        preferred_element_type=jnp.float32)
    h = jnp.maximum(h + b_ref[...], 0.0)
    lo_ref[...] = h[:, :F]
    hi_ref[...] = h[:, F:]


def _lin0_call(x, w, b):
    return pl.pallas_call(
        _lin0_body,
        grid=(N // BR,),
        in_specs=[
            pl.BlockSpec((BR, D), lambda i: (i, 0)),
            pl.BlockSpec((H, D), lambda i: (0, 0)),
            pl.BlockSpec((1, H), lambda i: (0, 0)),
        ],
        out_specs=[
            pl.BlockSpec((BR, F), lambda i: (i, 0)),
            pl.BlockSpec((BR, F), lambda i: (i, 0)),
        ],
        out_shape=[
            jax.ShapeDtypeStruct((N, F), jnp.float32),
            jax.ShapeDtypeStruct((N, F), jnp.float32),
        ],
    )(x, w, b)


def _comb1_body(alo_ref, ahi_ref, xlo_ref, xhi_ref, w_ref, lo_ref, hi_ref):
    agg = jnp.concatenate([alo_ref[...], ahi_ref[...]], axis=1)
    x0 = jnp.concatenate([xlo_ref[...], xhi_ref[...]], axis=1)
    u = (1.0 - ALPHA) * agg + ALPHA * x0
    o = (1.0 - BETA1) * u + BETA1 * jnp.dot(
        u, w_ref[...], preferred_element_type=jnp.float32)
    o = jnp.maximum(o, 0.0)
    lo_ref[...] = o[:, :F]
    hi_ref[...] = o[:, F:]


def _comb1_call(alo, ahi, xlo, xhi, w):
    return pl.pallas_call(
        _comb1_body,
        grid=(N // BR,),
        in_specs=[
            pl.BlockSpec((BR, F), lambda i: (i, 0)),
            pl.BlockSpec((BR, F), lambda i: (i, 0)),
            pl.BlockSpec((BR, F), lambda i: (i, 0)),
            pl.BlockSpec((BR, F), lambda i: (i, 0)),
            pl.BlockSpec((H, H), lambda i: (0, 0)),
        ],
        out_specs=[
            pl.BlockSpec((BR, F), lambda i: (i, 0)),
            pl.BlockSpec((BR, F), lambda i: (i, 0)),
        ],
        out_shape=[
            jax.ShapeDtypeStruct((N, F), jnp.float32),
            jax.ShapeDtypeStruct((N, F), jnp.float32),
        ],
    )(alo, ahi, xlo, xhi, w)


def _comb2_body(alo_ref, ahi_ref, xlo_ref, xhi_ref, w_ref, w1_ref, b1_ref, out_ref):
    agg = jnp.concatenate([alo_ref[...], ahi_ref[...]], axis=1)
    x0 = jnp.concatenate([xlo_ref[...], xhi_ref[...]], axis=1)
    u = (1.0 - ALPHA) * agg + ALPHA * x0
    o = (1.0 - BETA2) * u + BETA2 * jnp.dot(
        u, w_ref[...], preferred_element_type=jnp.float32)
    logits = lax.dot_general(o, w1_ref[...], (((1,), (1,)), ((), ())),
                             preferred_element_type=jnp.float32)
    out_ref[...] = logits + b1_ref[...]


def _comb2_call(alo, ahi, xlo, xhi, w, w1, b1):
    return pl.pallas_call(
        _comb2_body,
        grid=(N // BR,),
        in_specs=[
            pl.BlockSpec((BR, F), lambda i: (i, 0)),
            pl.BlockSpec((BR, F), lambda i: (i, 0)),
            pl.BlockSpec((BR, F), lambda i: (i, 0)),
            pl.BlockSpec((BR, F), lambda i: (i, 0)),
            pl.BlockSpec((H, H), lambda i: (0, 0)),
            pl.BlockSpec((OUT, H), lambda i: (0, 0)),
            pl.BlockSpec((1, OUT), lambda i: (0, 0)),
        ],
        out_specs=pl.BlockSpec((BR, OUT), lambda i: (i, 0)),
        out_shape=jax.ShapeDtypeStruct((N, OUT), jnp.float32),
    )(alo, ahi, xlo, xhi, w, w1, b1)


def kernel(x, edge_index, lin0_W, lin0_b, conv_W1, conv_W2, lin1_W, lin1_b):
    src = edge_index[0]
    dst = edge_index[1]
    h_lo, h_hi = _lin0_call(x, lin0_W, lin0_b.reshape(1, H))
    a1_lo, a1_hi = _segsum_sc(src, dst, h_lo, h_hi)
    o1_lo, o1_hi = _comb1_call(a1_lo, a1_hi, h_lo, h_hi, conv_W1)
    a2_lo, a2_hi = _segsum_sc(src, dst, o1_lo, o1_hi)
    return _comb2_call(a2_lo, a2_hi, h_lo, h_hi, conv_W2, lin1_W,
                       lin1_b.reshape(1, OUT))
